# baseline (speedup 1.0000x reference)
import functools

import jax
import jax.numpy as jnp
from jax.experimental import pallas as pl
from jax.experimental.pallas import tpu as pltpu
from jax.experimental.pallas import tpu_sc as plsc

_B, _C, _N = 4, 64, 4096
_NP = 1024
_K = 32
_R0, _R1 = 0.1, 0.2
_GW = 128


def _fps_body(coor_ref, nc_ref):
    x = coor_ref[:, 0, :]
    y = coor_ref[:, 1, :]
    z = coor_ref[:, 2, :]
    lanes = jax.lax.broadcasted_iota(jnp.int32, (_B, _N), 1)
    slots = jax.lax.broadcasted_iota(jnp.int32, (_B, _NP), 1)

    def body(i, st):
        dist, far, ax, ay, az = st
        sel = lanes == far
        cx = jnp.sum(jnp.where(sel, x, 0.0), axis=1, keepdims=True)
        cy = jnp.sum(jnp.where(sel, y, 0.0), axis=1, keepdims=True)
        cz = jnp.sum(jnp.where(sel, z, 0.0), axis=1, keepdims=True)
        ism = slots == i
        ax = jnp.where(ism, cx, ax)
        ay = jnp.where(ism, cy, ay)
        az = jnp.where(ism, cz, az)
        d = (x - cx) ** 2 + (y - cy) ** 2 + (z - cz) ** 2
        dist = jnp.minimum(dist, d)
        m = jnp.max(dist, axis=1, keepdims=True)
        far = jnp.min(jnp.where(dist == m, lanes, _N), axis=1, keepdims=True)
        return dist, far.astype(jnp.int32), ax, ay, az

    dist0 = jnp.full((_B, _N), 1e10, jnp.float32)
    far0 = jnp.zeros((_B, 1), jnp.int32)
    a0 = jnp.zeros((_B, _NP), jnp.float32)
    _, _, ax, ay, az = jax.lax.fori_loop(0, _NP, body, (dist0, far0, a0, a0, a0))
    nc_ref[:, 0, :] = ax
    nc_ref[:, 1, :] = ay
    nc_ref[:, 2, :] = az


def _fps(points_coor):
    return pl.pallas_call(
        _fps_body,
        out_shape=jax.ShapeDtypeStruct((_B, 3, _NP), jnp.float32),
    )(points_coor)


def _bq_body(M, r2, base, pts_ref, ct_ref, out_ref):
    b = pl.program_id(0)
    kio = jax.lax.broadcasted_iota(jnp.int32, (8, _K), 1)
    kvec = jax.lax.broadcasted_iota(jnp.int32, (8, _K, 1), 1)
    l128 = jax.lax.broadcasted_iota(jnp.int32, (8, 1, 128), 2)

    def body(sg, carry):
        c8 = ct_ref[0, pl.ds(sg * 8, 8), :]
        cx = c8[:, 0:1]
        cy = c8[:, 1:2]
        cz = c8[:, 2:3]
        acc = jnp.zeros((8, _K, 128), jnp.int32)
        eb = jnp.zeros((8, 1, 1), jnp.int32)
        for c in range(M // 128):
            sl = slice(c * 128, (c + 1) * 128)
            d2 = ((cx - pts_ref[0, 0:1, sl]) ** 2
                  + (cy - pts_ref[0, 1:2, sl]) ** 2
                  + (cz - pts_ref[0, 2:3, sl]) ** 2)
            w = (d2 <= r2).astype(jnp.int32)[:, None, :]
            s = 1
            while s < 128:
                w = w + jnp.where(l128 >= s, pltpu.roll(w, s, 2), 0)
                s *= 2
            m = kvec - eb
            acc = acc + (w <= m).astype(jnp.int32)
            eb = eb + w[:, :, 127:128]
        idx = jnp.sum(acc, axis=2)
        cnt = eb[:, :, 0]
        first = idx[:, 0:1]
        tile = jnp.where(kio < cnt, idx, first)
        out_ref[0, pl.ds(sg * 8, 8), :] = tile + b * base
        return carry

    jax.lax.fori_loop(0, _NP // 8, body, 0)


def _ball_query(cands, centers_t, M, r2, base):
    return pl.pallas_call(
        functools.partial(_bq_body, M, r2, base),
        grid=(_B,),
        in_specs=[
            pl.BlockSpec((1, 3, M), lambda b: (b, 0, 0)),
            pl.BlockSpec((1, _NP, 8), lambda b: (b, 0, 0)),
        ],
        out_specs=pl.BlockSpec((1, _NP, _K), lambda b: (b, 0, 0)),
        out_shape=jax.ShapeDtypeStruct((_B, _NP, _K), jnp.int32),
    )(cands, centers_t)


def _proj_body(fea_ref, cp_ref, ncp_ref, wf_ref, wc_ref, bias_ref,
               u_ref, v_ref):
    u_ref[...] = jnp.dot(
        fea_ref[0], wf_ref[...], preferred_element_type=jnp.float32
    ) + jnp.dot(cp_ref[0], wc_ref[...], preferred_element_type=jnp.float32)
    v_ref[0] = jnp.dot(
        ncp_ref[0], wc_ref[...], preferred_element_type=jnp.float32
    ) - bias_ref[...]


def _project(fea_t, coor_p, nc_p, wf_t, wc_t, bias, M, D):
    return pl.pallas_call(
        _proj_body,
        grid=(_B,),
        in_specs=[
            pl.BlockSpec((1, M, D), lambda b: (b, 0, 0)),
            pl.BlockSpec((1, M, 8), lambda b: (b, 0, 0)),
            pl.BlockSpec((1, _NP, 8), lambda b: (b, 0, 0)),
            pl.BlockSpec((D, 128), lambda b: (0, 0)),
            pl.BlockSpec((8, 128), lambda b: (0, 0)),
            pl.BlockSpec((1, 128), lambda b: (0, 0)),
        ],
        out_specs=[
            pl.BlockSpec((M, 128), lambda b: (b, 0)),
            pl.BlockSpec((1, _NP, 128), lambda b: (b, 0, 0)),
        ],
        out_shape=[
            jax.ShapeDtypeStruct((_B * M, 128), jnp.float32),
            jax.ShapeDtypeStruct((_B, _NP, 128), jnp.float32),
        ],
    )(fea_t, coor_p, nc_p, wf_t, wc_t, bias)


def _gather_rows(table, idx_flat, rows):
    mesh = plsc.VectorSubcoreMesh(core_axis_name="c", subcore_axis_name="s")

    @functools.partial(
        pl.kernel,
        out_type=jax.ShapeDtypeStruct((rows, 128), table.dtype),
        mesh=mesh,
    )
    def k(x_hbm, i_hbm, o_hbm):
        def body(i_vmem, o_vmem):
            pltpu.sync_copy(x_hbm.at[i_vmem.at[0]], o_vmem)

        pltpu.emit_pipeline(
            body,
            grid=(rows // _GW,),
            in_specs=[pl.BlockSpec((1, _GW), lambda i: (0, i))],
            out_specs=[pl.BlockSpec((_GW, 128), lambda i: (i, 0))],
            core_axis_name=("c", "s"),
            dimension_semantics=(pltpu.PARALLEL,),
        )(i_hbm, o_hbm)

    return k(table, idx_flat)


def _ln(x, g, be):
    m = jnp.mean(x, axis=-1, keepdims=True)
    v = jnp.mean((x - m) ** 2, axis=-1, keepdims=True)
    return (x - m) / jnp.sqrt(v + 1e-5) * g + be


def _pool_body(g_ref, v_ref, gam_ref, bet_ref, out_ref):
    blk = g_ref.shape[0] // _K
    x = g_ref[...].reshape(blk, _K, 128) - v_ref[0].reshape(blk, 1, 128)
    x = jnp.maximum(_ln(x, gam_ref[...], bet_ref[...]), 0.0)
    out_ref[0] = jnp.max(x, axis=1)


def _pool(gathered, v, gamma, beta, blk=256):
    nb = _NP // blk
    return pl.pallas_call(
        _pool_body,
        grid=(_B, nb),
        in_specs=[
            pl.BlockSpec((blk * _K, 128), lambda b, j: (b * nb + j, 0)),
            pl.BlockSpec((1, blk, 128), lambda b, j: (b, j, 0)),
            pl.BlockSpec((1, 128), lambda b, j: (0, 0)),
            pl.BlockSpec((1, 128), lambda b, j: (0, 0)),
        ],
        out_specs=pl.BlockSpec((1, blk, 128), lambda b, j: (b, j, 0)),
        out_shape=jax.ShapeDtypeStruct((_B, _NP, 128), jnp.float32),
    )(gathered, v, gamma, beta)


def _pw_body(y_ref, id_ref, w1_ref, b1_ref, g1_ref, be1_ref,
             w2_ref, b2_ref, g2_ref, be2_ref, out_ref):
    y = y_ref[0]
    h = jnp.dot(y, w1_ref[...], preferred_element_type=jnp.float32) + b1_ref[...]
    h = jnp.maximum(_ln(h, g1_ref[...], be1_ref[...]), 0.0)
    o = jnp.dot(h, w2_ref[...], preferred_element_type=jnp.float32) + b2_ref[...]
    o = _ln(o, g2_ref[...], be2_ref[...])
    out_ref[0] = jnp.maximum(o + id_ref[0], 0.0)


def _pointwise(y, ident, w1t, b1, g1, be1, w2t, b2, g2, be2):
    return pl.pallas_call(
        _pw_body,
        grid=(_B,),
        in_specs=[
            pl.BlockSpec((1, _NP, 128), lambda b: (b, 0, 0)),
            pl.BlockSpec((1, _NP, 128), lambda b: (b, 0, 0)),
            pl.BlockSpec((128, 512), lambda b: (0, 0)),
            pl.BlockSpec((1, 512), lambda b: (0, 0)),
            pl.BlockSpec((1, 512), lambda b: (0, 0)),
            pl.BlockSpec((1, 512), lambda b: (0, 0)),
            pl.BlockSpec((512, 128), lambda b: (0, 0)),
            pl.BlockSpec((1, 128), lambda b: (0, 0)),
            pl.BlockSpec((1, 128), lambda b: (0, 0)),
            pl.BlockSpec((1, 128), lambda b: (0, 0)),
        ],
        out_specs=pl.BlockSpec((1, _NP, 128), lambda b: (b, 0, 0)),
        out_shape=jax.ShapeDtypeStruct((_B, _NP, 128), jnp.float32),
    )(y, ident, w1t, b1, g1, be1, w2t, b2, g2, be2)


def kernel(points_coor, points_fea, points_padding, W_sa, b_sa, g_sa, be_sa,
           W_la, b_la, g_la, be_la, W_pw1, b_pw1, g_pw1, be_pw1,
           W_pw2, b_pw2, g_pw2, be_pw2):
    f32 = jnp.float32

    def pad8(x3):
        return jnp.pad(x3, ((0, 0), (0, 0), (0, 5)))

    new_coor_t = _fps(points_coor)
    nc_p = pad8(jnp.transpose(new_coor_t, (0, 2, 1)))

    gidx1 = _ball_query(points_coor, nc_p, _N, _R0 * _R0, _N)
    gidx2 = _ball_query(new_coor_t, nc_p, _NP, _R1 * _R1, _NP)

    fea_t = jnp.transpose(points_fea, (0, 2, 1))
    coor_p = pad8(jnp.transpose(points_coor, (0, 2, 1)))
    wf1 = jnp.transpose(W_sa[:, :_C])
    wc1 = jnp.pad(jnp.transpose(W_sa[:, _C:]) / _R0, ((0, 5), (0, 0)))
    u1, v1 = _project(fea_t, coor_p, nc_p, wf1, wc1,
                      (-b_sa).reshape(1, 128).astype(f32), _N, _C)

    g1 = _gather_rows(u1, gidx1.reshape(1, _B * _NP * _K), _B * _NP * _K)
    new_fea = _pool(g1, v1, g_sa.reshape(1, 128), be_sa.reshape(1, 128))

    wf2 = jnp.transpose(W_la[:, : 2 * _C])
    wc2 = jnp.pad(jnp.transpose(W_la[:, 2 * _C :]) / _R1, ((0, 5), (0, 0)))
    u2, v2 = _project(new_fea, nc_p, nc_p, wf2, wc2,
                      (-b_la).reshape(1, 128).astype(f32), _NP, 2 * _C)
    g2 = _gather_rows(u2, gidx2.reshape(1, _B * _NP * _K), _B * _NP * _K)
    y = _pool(g2, v2, g_la.reshape(1, 128), be_la.reshape(1, 128))

    out = _pointwise(
        y, new_fea,
        jnp.transpose(W_pw1), b_pw1.reshape(1, 512),
        g_pw1.reshape(1, 512), be_pw1.reshape(1, 512),
        jnp.transpose(W_pw2), b_pw2.reshape(1, 128),
        g_pw2.reshape(1, 128), be_pw2.reshape(1, 128),
    )
    out_fea = jnp.transpose(out, (0, 2, 1))
    new_mask = jnp.zeros((_B, _NP), dtype=bool)
    return (new_coor_t, out_fea, new_mask)


# speedup vs baseline: 4.1577x; 4.1577x over previous
import functools

import jax
import jax.numpy as jnp
from jax.experimental import pallas as pl
from jax.experimental.pallas import tpu as pltpu
from jax.experimental.pallas import tpu_sc as plsc

_B, _C, _N = 4, 64, 4096
_NP = 1024
_K = 32
_R0, _R1 = 0.1, 0.2
_GW = 128


def _fps_body(coor_ref, gidx_ref, nc_ref):
    H = _N // 2
    x = coor_ref[0]
    y = coor_ref[1]
    z = coor_ref[2]
    gidx = gidx_ref[...]
    slots = jax.lax.broadcasted_iota(jnp.int32, (8, _NP), 1)

    def half_comb(a, op):
        return op(a, pltpu.roll(a, 4, 0))

    def body(i, st):
        dist, far8, ax, ay, az = st
        sel = gidx == far8
        cx = half_comb(
            jnp.sum(jnp.where(sel, x, 0.0), axis=1, keepdims=True), jnp.add)
        cy = half_comb(
            jnp.sum(jnp.where(sel, y, 0.0), axis=1, keepdims=True), jnp.add)
        cz = half_comb(
            jnp.sum(jnp.where(sel, z, 0.0), axis=1, keepdims=True), jnp.add)
        ism = slots == i
        ax = jnp.where(ism, cx, ax)
        ay = jnp.where(ism, cy, ay)
        az = jnp.where(ism, cz, az)
        d = (x - cx) ** 2 + (y - cy) ** 2 + (z - cz) ** 2
        dist = jnp.minimum(dist, d)
        m8 = half_comb(jnp.max(dist, axis=1, keepdims=True), jnp.maximum)
        far8 = half_comb(
            jnp.min(jnp.where(dist == m8, gidx, _N), axis=1, keepdims=True),
            jnp.minimum).astype(jnp.int32)
        return dist, far8, ax, ay, az

    dist0 = jnp.full((8, H), 1e10, jnp.float32)
    far0 = jnp.zeros((8, 1), jnp.int32)
    a0 = jnp.zeros((8, _NP), jnp.float32)
    _, _, ax, ay, az = jax.lax.fori_loop(0, _NP, body, (dist0, far0, a0, a0, a0))
    nc_ref[:, 0, :] = ax[0:4]
    nc_ref[:, 1, :] = ay[0:4]
    nc_ref[:, 2, :] = az[0:4]


def _fps(points_coor):
    H = _N // 2
    packed = jnp.concatenate(
        [points_coor[:, :, :H], points_coor[:, :, H:]], axis=0
    ).transpose(1, 0, 2)
    gidx = (jnp.arange(H, dtype=jnp.int32)[None, :]
            + jnp.where(jnp.arange(8, dtype=jnp.int32)[:, None] >= 4, H, 0))
    return pl.pallas_call(
        _fps_body,
        out_shape=jax.ShapeDtypeStruct((_B, 3, _NP), jnp.float32),
    )(packed, gidx)


def _bq_body(M, r2, base, pts_ref, ct_ref, out_ref):
    b = pl.program_id(0)
    xs = pts_ref[0, 0:1, :]
    ys = pts_ref[0, 1:2, :]
    zs = pts_ref[0, 2:3, :]
    lanes = jax.lax.broadcasted_iota(jnp.int32, (8, M), 1)

    def body(sg, carry):
        c8 = ct_ref[0, pl.ds(sg * 8, 8), :]
        cx = c8[:, 0:1]
        cy = c8[:, 1:2]
        cz = c8[:, 2:3]
        d2 = (cx - xs) ** 2 + (cy - ys) ** 2 + (cz - zs) ** 2
        w = (d2 <= r2).astype(jnp.int32)
        s = 1
        while s < M:
            w = w + jnp.where(lanes >= s, pltpu.roll(w, s, 1), 0)
            s *= 2
        cnt = w[:, M - 1 : M]
        cols = [jnp.sum((w <= k).astype(jnp.int32), axis=1, keepdims=True)
                for k in range(_K)]
        idx = jnp.concatenate(cols, axis=1)
        kio = jax.lax.broadcasted_iota(jnp.int32, (8, _K), 1)
        tile = jnp.where(kio < cnt, idx, cols[0])
        out_ref[0, pl.ds(sg * 8, 8), :] = tile + b * base
        return carry

    jax.lax.fori_loop(0, _NP // 8, body, 0)


def _ball_query(cands, centers_t, M, r2, base):
    return pl.pallas_call(
        functools.partial(_bq_body, M, r2, base),
        grid=(_B,),
        in_specs=[
            pl.BlockSpec((1, 3, M), lambda b: (b, 0, 0)),
            pl.BlockSpec((1, _NP, 8), lambda b: (b, 0, 0)),
        ],
        out_specs=pl.BlockSpec((1, _NP, _K), lambda b: (b, 0, 0)),
        out_shape=jax.ShapeDtypeStruct((_B, _NP, _K), jnp.int32),
    )(cands, centers_t)


def _proj_body(fea_ref, cp_ref, ncp_ref, wf_ref, wc_ref, bias_ref,
               u_ref, v_ref):
    u_ref[...] = jnp.dot(
        fea_ref[0], wf_ref[...], preferred_element_type=jnp.float32
    ) + jnp.dot(cp_ref[0], wc_ref[...], preferred_element_type=jnp.float32)
    v_ref[0] = jnp.dot(
        ncp_ref[0], wc_ref[...], preferred_element_type=jnp.float32
    ) - bias_ref[...]


def _project(fea_t, coor_p, nc_p, wf_t, wc_t, bias, M, D):
    return pl.pallas_call(
        _proj_body,
        grid=(_B,),
        in_specs=[
            pl.BlockSpec((1, M, D), lambda b: (b, 0, 0)),
            pl.BlockSpec((1, M, 8), lambda b: (b, 0, 0)),
            pl.BlockSpec((1, _NP, 8), lambda b: (b, 0, 0)),
            pl.BlockSpec((D, 128), lambda b: (0, 0)),
            pl.BlockSpec((8, 128), lambda b: (0, 0)),
            pl.BlockSpec((1, 128), lambda b: (0, 0)),
        ],
        out_specs=[
            pl.BlockSpec((M, 128), lambda b: (b, 0)),
            pl.BlockSpec((1, _NP, 128), lambda b: (b, 0, 0)),
        ],
        out_shape=[
            jax.ShapeDtypeStruct((_B * M, 128), jnp.float32),
            jax.ShapeDtypeStruct((_B, _NP, 128), jnp.float32),
        ],
    )(fea_t, coor_p, nc_p, wf_t, wc_t, bias)


def _gather_rows(table, idx_flat, rows):
    mesh = plsc.VectorSubcoreMesh(core_axis_name="c", subcore_axis_name="s")

    @functools.partial(
        pl.kernel,
        out_type=jax.ShapeDtypeStruct((rows, 128), table.dtype),
        mesh=mesh,
    )
    def k(x_hbm, i_hbm, o_hbm):
        def body(i_vmem, o_vmem):
            pltpu.sync_copy(x_hbm.at[i_vmem.at[0]], o_vmem)

        pltpu.emit_pipeline(
            body,
            grid=(rows // _GW,),
            in_specs=[pl.BlockSpec((1, _GW), lambda i: (0, i))],
            out_specs=[pl.BlockSpec((_GW, 128), lambda i: (i, 0))],
            core_axis_name=("c", "s"),
            dimension_semantics=(pltpu.PARALLEL,),
        )(i_hbm, o_hbm)

    return k(table, idx_flat)


def _ln(x, g, be):
    m = jnp.mean(x, axis=-1, keepdims=True)
    v = jnp.mean((x - m) ** 2, axis=-1, keepdims=True)
    return (x - m) / jnp.sqrt(v + 1e-5) * g + be


def _pool_body(g_ref, v_ref, gam_ref, bet_ref, out_ref):
    blk = g_ref.shape[0] // _K
    x = g_ref[...].reshape(blk, _K, 128) - v_ref[0].reshape(blk, 1, 128)
    x = jnp.maximum(_ln(x, gam_ref[...], bet_ref[...]), 0.0)
    out_ref[0] = jnp.max(x, axis=1)


def _pool(gathered, v, gamma, beta, blk=256):
    nb = _NP // blk
    return pl.pallas_call(
        _pool_body,
        grid=(_B, nb),
        in_specs=[
            pl.BlockSpec((blk * _K, 128), lambda b, j: (b * nb + j, 0)),
            pl.BlockSpec((1, blk, 128), lambda b, j: (b, j, 0)),
            pl.BlockSpec((1, 128), lambda b, j: (0, 0)),
            pl.BlockSpec((1, 128), lambda b, j: (0, 0)),
        ],
        out_specs=pl.BlockSpec((1, blk, 128), lambda b, j: (b, j, 0)),
        out_shape=jax.ShapeDtypeStruct((_B, _NP, 128), jnp.float32),
    )(gathered, v, gamma, beta)


def _pw_body(y_ref, id_ref, w1_ref, b1_ref, g1_ref, be1_ref,
             w2_ref, b2_ref, g2_ref, be2_ref, out_ref):
    y = y_ref[0]
    h = jnp.dot(y, w1_ref[...], preferred_element_type=jnp.float32) + b1_ref[...]
    h = jnp.maximum(_ln(h, g1_ref[...], be1_ref[...]), 0.0)
    o = jnp.dot(h, w2_ref[...], preferred_element_type=jnp.float32) + b2_ref[...]
    o = _ln(o, g2_ref[...], be2_ref[...])
    out_ref[0] = jnp.maximum(o + id_ref[0], 0.0)


def _pointwise(y, ident, w1t, b1, g1, be1, w2t, b2, g2, be2):
    return pl.pallas_call(
        _pw_body,
        grid=(_B,),
        in_specs=[
            pl.BlockSpec((1, _NP, 128), lambda b: (b, 0, 0)),
            pl.BlockSpec((1, _NP, 128), lambda b: (b, 0, 0)),
            pl.BlockSpec((128, 512), lambda b: (0, 0)),
            pl.BlockSpec((1, 512), lambda b: (0, 0)),
            pl.BlockSpec((1, 512), lambda b: (0, 0)),
            pl.BlockSpec((1, 512), lambda b: (0, 0)),
            pl.BlockSpec((512, 128), lambda b: (0, 0)),
            pl.BlockSpec((1, 128), lambda b: (0, 0)),
            pl.BlockSpec((1, 128), lambda b: (0, 0)),
            pl.BlockSpec((1, 128), lambda b: (0, 0)),
        ],
        out_specs=pl.BlockSpec((1, _NP, 128), lambda b: (b, 0, 0)),
        out_shape=jax.ShapeDtypeStruct((_B, _NP, 128), jnp.float32),
    )(y, ident, w1t, b1, g1, be1, w2t, b2, g2, be2)


def kernel(points_coor, points_fea, points_padding, W_sa, b_sa, g_sa, be_sa,
           W_la, b_la, g_la, be_la, W_pw1, b_pw1, g_pw1, be_pw1,
           W_pw2, b_pw2, g_pw2, be_pw2):
    f32 = jnp.float32

    def pad8(x3):
        return jnp.pad(x3, ((0, 0), (0, 0), (0, 5)))

    new_coor_t = _fps(points_coor)
    nc_p = pad8(jnp.transpose(new_coor_t, (0, 2, 1)))

    gidx1 = _ball_query(points_coor, nc_p, _N, _R0 * _R0, _N)
    gidx2 = _ball_query(new_coor_t, nc_p, _NP, _R1 * _R1, _NP)

    fea_t = jnp.transpose(points_fea, (0, 2, 1))
    coor_p = pad8(jnp.transpose(points_coor, (0, 2, 1)))
    wf1 = jnp.transpose(W_sa[:, :_C])
    wc1 = jnp.pad(jnp.transpose(W_sa[:, _C:]) / _R0, ((0, 5), (0, 0)))
    u1, v1 = _project(fea_t, coor_p, nc_p, wf1, wc1,
                      (-b_sa).reshape(1, 128).astype(f32), _N, _C)

    g1 = _gather_rows(u1, gidx1.reshape(1, _B * _NP * _K), _B * _NP * _K)
    new_fea = _pool(g1, v1, g_sa.reshape(1, 128), be_sa.reshape(1, 128))

    wf2 = jnp.transpose(W_la[:, : 2 * _C])
    wc2 = jnp.pad(jnp.transpose(W_la[:, 2 * _C :]) / _R1, ((0, 5), (0, 0)))
    u2, v2 = _project(new_fea, nc_p, nc_p, wf2, wc2,
                      (-b_la).reshape(1, 128).astype(f32), _NP, 2 * _C)
    g2 = _gather_rows(u2, gidx2.reshape(1, _B * _NP * _K), _B * _NP * _K)
    y = _pool(g2, v2, g_la.reshape(1, 128), be_la.reshape(1, 128))

    out = _pointwise(
        y, new_fea,
        jnp.transpose(W_pw1), b_pw1.reshape(1, 512),
        g_pw1.reshape(1, 512), be_pw1.reshape(1, 512),
        jnp.transpose(W_pw2), b_pw2.reshape(1, 128),
        g_pw2.reshape(1, 128), be_pw2.reshape(1, 128),
    )
    out_fea = jnp.transpose(out, (0, 2, 1))
    new_mask = jnp.zeros((_B, _NP), dtype=bool)
    return (new_coor_t, out_fea, new_mask)


# speedup vs baseline: 5.5769x; 1.3413x over previous
import functools

import jax
import jax.numpy as jnp
from jax.experimental import pallas as pl
from jax.experimental.pallas import tpu as pltpu
from jax.experimental.pallas import tpu_sc as plsc

_B, _C, _N = 4, 64, 4096
_NP = 1024
_K = 32
_R0, _R1 = 0.1, 0.2
_GW = 128


def _fps_body(coor_ref, gidx_ref, nc_ref):
    H = _N // 2
    x = coor_ref[0]
    y = coor_ref[1]
    z = coor_ref[2]
    gidx = gidx_ref[...]
    off = jnp.where(
        jax.lax.broadcasted_iota(jnp.int32, (8, 1), 0) >= 4, H, 0)
    slots = jax.lax.broadcasted_iota(jnp.int32, (8, _NP), 1)

    def half_comb(a, op):
        return op(a, pltpu.roll(a, 4, 0))

    def body(i, st):
        dist, far8, ax, ay, az = st
        sel = gidx == far8
        cx = half_comb(
            jnp.sum(jnp.where(sel, x, 0.0), axis=1, keepdims=True), jnp.add)
        cy = half_comb(
            jnp.sum(jnp.where(sel, y, 0.0), axis=1, keepdims=True), jnp.add)
        cz = half_comb(
            jnp.sum(jnp.where(sel, z, 0.0), axis=1, keepdims=True), jnp.add)
        ism = slots == i
        ax = jnp.where(ism, cx, ax)
        ay = jnp.where(ism, cy, ay)
        az = jnp.where(ism, cz, az)
        d = (x - cx) ** 2 + (y - cy) ** 2 + (z - cz) ** 2
        dist = jnp.minimum(dist, d)
        mrow = jnp.max(dist, axis=1, keepdims=True)
        irow = (jnp.argmax(dist, axis=1).astype(jnp.int32).reshape(8, 1)
                + off)
        mp = pltpu.roll(mrow, 4, 0)
        ip = pltpu.roll(irow, 4, 0)
        better = (mp > mrow) | ((mp == mrow) & (ip < irow))
        far8 = jnp.where(better, ip, irow)
        return dist, far8, ax, ay, az

    dist0 = jnp.full((8, H), 1e10, jnp.float32)
    far0 = jnp.zeros((8, 1), jnp.int32)
    a0 = jnp.zeros((8, _NP), jnp.float32)
    _, _, ax, ay, az = jax.lax.fori_loop(0, _NP, body, (dist0, far0, a0, a0, a0))
    nc_ref[:, 0, :] = ax[0:4]
    nc_ref[:, 1, :] = ay[0:4]
    nc_ref[:, 2, :] = az[0:4]


def _fps(points_coor):
    H = _N // 2
    packed = jnp.concatenate(
        [points_coor[:, :, :H], points_coor[:, :, H:]], axis=0
    ).transpose(1, 0, 2)
    gidx = (jnp.arange(H, dtype=jnp.int32)[None, :]
            + jnp.where(jnp.arange(8, dtype=jnp.int32)[:, None] >= 4, H, 0))
    return pl.pallas_call(
        _fps_body,
        out_shape=jax.ShapeDtypeStruct((_B, 3, _NP), jnp.float32),
    )(packed, gidx)


def _bq_body(M, r2, base, pts_ref, ct_ref, out_ref):
    b = pl.program_id(0)
    G = 16
    xs = pts_ref[0, 0:1, :]
    ys = pts_ref[0, 1:2, :]
    zs = pts_ref[0, 2:3, :]
    lanes = jax.lax.broadcasted_iota(jnp.int32, (G, M), 1)

    def body(sg, carry):
        c8 = ct_ref[0, pl.ds(sg * G, G), :]
        cx = c8[:, 0:1]
        cy = c8[:, 1:2]
        cz = c8[:, 2:3]
        d2 = (cx - xs) ** 2 + (cy - ys) ** 2 + (cz - zs) ** 2
        w = (d2 <= r2).astype(jnp.int32)
        s = 1
        while s < M:
            w = w + jnp.where(lanes >= s, pltpu.roll(w, s, 1), 0)
            s *= 2
        cnt = w[:, M - 1 : M]
        cols = [jnp.sum((w <= k).astype(jnp.int32), axis=1, keepdims=True)
                for k in range(_K)]
        idx = jnp.concatenate(cols, axis=1)
        kio = jax.lax.broadcasted_iota(jnp.int32, (G, _K), 1)
        tile = jnp.where(kio < cnt, idx, cols[0])
        out_ref[0, pl.ds(sg * G, G), :] = tile + b * base
        return carry

    jax.lax.fori_loop(0, _NP // G, body, 0)


def _ball_query(cands, centers_t, M, r2, base):
    return pl.pallas_call(
        functools.partial(_bq_body, M, r2, base),
        grid=(_B,),
        in_specs=[
            pl.BlockSpec((1, 3, M), lambda b: (b, 0, 0)),
            pl.BlockSpec((1, _NP, 8), lambda b: (b, 0, 0)),
        ],
        out_specs=pl.BlockSpec((1, _NP, _K), lambda b: (b, 0, 0)),
        out_shape=jax.ShapeDtypeStruct((_B, _NP, _K), jnp.int32),
    )(cands, centers_t)


def _proj_body(fea_ref, cp_ref, ncp_ref, wf_ref, wc_ref, bias_ref,
               u_ref, v_ref):
    u_ref[...] = jnp.dot(
        fea_ref[0], wf_ref[...], preferred_element_type=jnp.float32
    ) + jnp.dot(cp_ref[0], wc_ref[...], preferred_element_type=jnp.float32)
    v_ref[0] = jnp.dot(
        ncp_ref[0], wc_ref[...], preferred_element_type=jnp.float32
    ) - bias_ref[...]


def _project(fea_t, coor_p, nc_p, wf_t, wc_t, bias, M, D):
    return pl.pallas_call(
        _proj_body,
        grid=(_B,),
        in_specs=[
            pl.BlockSpec((1, M, D), lambda b: (b, 0, 0)),
            pl.BlockSpec((1, M, 8), lambda b: (b, 0, 0)),
            pl.BlockSpec((1, _NP, 8), lambda b: (b, 0, 0)),
            pl.BlockSpec((D, 128), lambda b: (0, 0)),
            pl.BlockSpec((8, 128), lambda b: (0, 0)),
            pl.BlockSpec((1, 128), lambda b: (0, 0)),
        ],
        out_specs=[
            pl.BlockSpec((M, 128), lambda b: (b, 0)),
            pl.BlockSpec((1, _NP, 128), lambda b: (b, 0, 0)),
        ],
        out_shape=[
            jax.ShapeDtypeStruct((_B * M, 128), jnp.float32),
            jax.ShapeDtypeStruct((_B, _NP, 128), jnp.float32),
        ],
    )(fea_t, coor_p, nc_p, wf_t, wc_t, bias)


def _gather_rows(table, idx_flat, rows):
    mesh = plsc.VectorSubcoreMesh(core_axis_name="c", subcore_axis_name="s")

    @functools.partial(
        pl.kernel,
        out_type=jax.ShapeDtypeStruct((rows, 128), table.dtype),
        mesh=mesh,
    )
    def k(x_hbm, i_hbm, o_hbm):
        def body(i_vmem, o_vmem):
            pltpu.sync_copy(x_hbm.at[i_vmem.at[0]], o_vmem)

        pltpu.emit_pipeline(
            body,
            grid=(rows // _GW,),
            in_specs=[pl.BlockSpec((1, _GW), lambda i: (0, i))],
            out_specs=[pl.BlockSpec((_GW, 128), lambda i: (i, 0))],
            core_axis_name=("c", "s"),
            dimension_semantics=(pltpu.PARALLEL,),
        )(i_hbm, o_hbm)

    return k(table, idx_flat)


def _ln(x, g, be):
    m = jnp.mean(x, axis=-1, keepdims=True)
    v = jnp.mean((x - m) ** 2, axis=-1, keepdims=True)
    return (x - m) / jnp.sqrt(v + 1e-5) * g + be


def _pool_body(g_ref, v_ref, gam_ref, bet_ref, out_ref):
    blk = g_ref.shape[0] // _K
    x = g_ref[...].reshape(blk, _K, 128) - v_ref[0].reshape(blk, 1, 128)
    x = jnp.maximum(_ln(x, gam_ref[...], bet_ref[...]), 0.0)
    out_ref[0] = jnp.max(x, axis=1)


def _pool(gathered, v, gamma, beta, blk=256):
    nb = _NP // blk
    return pl.pallas_call(
        _pool_body,
        grid=(_B, nb),
        in_specs=[
            pl.BlockSpec((blk * _K, 128), lambda b, j: (b * nb + j, 0)),
            pl.BlockSpec((1, blk, 128), lambda b, j: (b, j, 0)),
            pl.BlockSpec((1, 128), lambda b, j: (0, 0)),
            pl.BlockSpec((1, 128), lambda b, j: (0, 0)),
        ],
        out_specs=pl.BlockSpec((1, blk, 128), lambda b, j: (b, j, 0)),
        out_shape=jax.ShapeDtypeStruct((_B, _NP, 128), jnp.float32),
    )(gathered, v, gamma, beta)


def _pw_body(y_ref, id_ref, w1_ref, b1_ref, g1_ref, be1_ref,
             w2_ref, b2_ref, g2_ref, be2_ref, out_ref):
    y = y_ref[0]
    h = jnp.dot(y, w1_ref[...], preferred_element_type=jnp.float32) + b1_ref[...]
    h = jnp.maximum(_ln(h, g1_ref[...], be1_ref[...]), 0.0)
    o = jnp.dot(h, w2_ref[...], preferred_element_type=jnp.float32) + b2_ref[...]
    o = _ln(o, g2_ref[...], be2_ref[...])
    out_ref[0] = jnp.maximum(o + id_ref[0], 0.0)


def _pointwise(y, ident, w1t, b1, g1, be1, w2t, b2, g2, be2):
    return pl.pallas_call(
        _pw_body,
        grid=(_B,),
        in_specs=[
            pl.BlockSpec((1, _NP, 128), lambda b: (b, 0, 0)),
            pl.BlockSpec((1, _NP, 128), lambda b: (b, 0, 0)),
            pl.BlockSpec((128, 512), lambda b: (0, 0)),
            pl.BlockSpec((1, 512), lambda b: (0, 0)),
            pl.BlockSpec((1, 512), lambda b: (0, 0)),
            pl.BlockSpec((1, 512), lambda b: (0, 0)),
            pl.BlockSpec((512, 128), lambda b: (0, 0)),
            pl.BlockSpec((1, 128), lambda b: (0, 0)),
            pl.BlockSpec((1, 128), lambda b: (0, 0)),
            pl.BlockSpec((1, 128), lambda b: (0, 0)),
        ],
        out_specs=pl.BlockSpec((1, _NP, 128), lambda b: (b, 0, 0)),
        out_shape=jax.ShapeDtypeStruct((_B, _NP, 128), jnp.float32),
    )(y, ident, w1t, b1, g1, be1, w2t, b2, g2, be2)


def kernel(points_coor, points_fea, points_padding, W_sa, b_sa, g_sa, be_sa,
           W_la, b_la, g_la, be_la, W_pw1, b_pw1, g_pw1, be_pw1,
           W_pw2, b_pw2, g_pw2, be_pw2):
    f32 = jnp.float32

    def pad8(x3):
        return jnp.pad(x3, ((0, 0), (0, 0), (0, 5)))

    new_coor_t = _fps(points_coor)
    nc_p = pad8(jnp.transpose(new_coor_t, (0, 2, 1)))

    gidx1 = _ball_query(points_coor, nc_p, _N, _R0 * _R0, _N)
    gidx2 = _ball_query(new_coor_t, nc_p, _NP, _R1 * _R1, _NP)

    fea_t = jnp.transpose(points_fea, (0, 2, 1))
    coor_p = pad8(jnp.transpose(points_coor, (0, 2, 1)))
    wf1 = jnp.transpose(W_sa[:, :_C])
    wc1 = jnp.pad(jnp.transpose(W_sa[:, _C:]) / _R0, ((0, 5), (0, 0)))
    u1, v1 = _project(fea_t, coor_p, nc_p, wf1, wc1,
                      (-b_sa).reshape(1, 128).astype(f32), _N, _C)

    g1 = _gather_rows(u1, gidx1.reshape(1, _B * _NP * _K), _B * _NP * _K)
    new_fea = _pool(g1, v1, g_sa.reshape(1, 128), be_sa.reshape(1, 128))

    wf2 = jnp.transpose(W_la[:, : 2 * _C])
    wc2 = jnp.pad(jnp.transpose(W_la[:, 2 * _C :]) / _R1, ((0, 5), (0, 0)))
    u2, v2 = _project(new_fea, nc_p, nc_p, wf2, wc2,
                      (-b_la).reshape(1, 128).astype(f32), _NP, 2 * _C)
    g2 = _gather_rows(u2, gidx2.reshape(1, _B * _NP * _K), _B * _NP * _K)
    y = _pool(g2, v2, g_la.reshape(1, 128), be_la.reshape(1, 128))

    out = _pointwise(
        y, new_fea,
        jnp.transpose(W_pw1), b_pw1.reshape(1, 512),
        g_pw1.reshape(1, 512), be_pw1.reshape(1, 512),
        jnp.transpose(W_pw2), b_pw2.reshape(1, 128),
        g_pw2.reshape(1, 128), be_pw2.reshape(1, 128),
    )
    out_fea = jnp.transpose(out, (0, 2, 1))
    new_mask = jnp.zeros((_B, _NP), dtype=bool)
    return (new_coor_t, out_fea, new_mask)
